# Initial kernel scaffold; baseline (speedup 1.0000x reference)
import jax
import jax.numpy as jnp
from jax.experimental import pallas as pl


def kernel(x, edge_index, W1, b1, W2, b2, Wf1, bf1, Wf2, bf2):
    raise NotImplementedError("write your pallas kernel here")


# speedup vs baseline: 115.4625x; 115.4625x over previous
import functools

import jax
import jax.numpy as jnp
from jax import lax
from jax.experimental import pallas as pl
from jax.experimental.pallas import tpu as pltpu
from jax.experimental.pallas import tpu_sc as plsc

N = 50000
E = 1600000
CONV = 32
HID = 512
OUT = 256
SLOPE = 0.01

NC = 2
NS = 16
NW = NC * NS
PER_TILE = E // NW
CHUNK = 2000

ROWS = 125
COLS = 400
SUB = 8
LANES = N // SUB

_mesh = plsc.VectorSubcoreMesh(core_axis_name="c", subcore_axis_name="s")


def _zero_vmem(acc_v):
    zero16 = jnp.zeros((16,), jnp.float32)

    def zbody(i, carry):
        acc_v[pl.ds(pl.multiple_of(i * 16, 16), 16)] = zero16
        return carry

    lax.fori_loop(0, N // 16, zbody, 0)


def _deg_body(dst_hbm, out_hbm, acc_v, idx_v):
    wid = lax.axis_index("s") * NC + lax.axis_index("c")
    _zero_vmem(acc_v)
    base = wid * PER_TILE
    ones16 = jnp.ones((16,), jnp.float32)

    def chunk_body(k, carry):
        pltpu.sync_copy(dst_hbm.at[pl.ds(base + k * CHUNK, CHUNK)], idx_v)

        def gbody(j, c2):
            d = idx_v[pl.ds(pl.multiple_of(j * 16, 16), 16)]
            plsc.addupdate_scatter(acc_v, [d], ones16)
            return c2

        lax.fori_loop(0, CHUNK // 16, gbody, 0)
        return carry

    lax.fori_loop(0, PER_TILE // CHUNK, chunk_body, 0)
    pltpu.sync_copy(acc_v, out_hbm.at[wid])


_sc_params = pltpu.CompilerParams(needs_layout_passes=False)

_deg_kernel = functools.partial(
    pl.kernel,
    mesh=_mesh,
    out_type=jax.ShapeDtypeStruct((NW, N), jnp.float32),
    compiler_params=_sc_params,
    scratch_types=[
        pltpu.VMEM((N,), jnp.float32),
        pltpu.VMEM((CHUNK,), jnp.int32),
    ],
)(_deg_body)


def _edge_body(src_hbm, dst_hbm, u_hbm, out_hbm, u_v, acc_v, src_v, dst_v):
    wid = lax.axis_index("s") * NC + lax.axis_index("c")
    _zero_vmem(acc_v)
    pltpu.sync_copy(u_hbm, u_v)
    base = wid * PER_TILE

    def chunk_body(k, carry):
        pltpu.sync_copy(src_hbm.at[pl.ds(base + k * CHUNK, CHUNK)], src_v)
        pltpu.sync_copy(dst_hbm.at[pl.ds(base + k * CHUNK, CHUNK)], dst_v)

        def gbody(j, c2):
            sl = pl.ds(pl.multiple_of(j * 16, 16), 16)
            s = src_v[sl]
            d = dst_v[sl]
            vals = plsc.load_gather(u_v, [s])
            plsc.addupdate_scatter(acc_v, [d], vals)
            return c2

        lax.fori_loop(0, CHUNK // 16, gbody, 0)
        return carry

    lax.fori_loop(0, PER_TILE // CHUNK, chunk_body, 0)
    pltpu.sync_copy(acc_v, out_hbm.at[wid])


_edge_kernel = functools.partial(
    pl.kernel,
    mesh=_mesh,
    out_type=jax.ShapeDtypeStruct((NW, N), jnp.float32),
    compiler_params=_sc_params,
    scratch_types=[
        pltpu.VMEM((N,), jnp.float32),
        pltpu.VMEM((N,), jnp.float32),
        pltpu.VMEM((CHUNK,), jnp.int32),
        pltpu.VMEM((CHUNK,), jnp.int32),
    ],
)(_edge_body)


def _lrelu(x):
    return jnp.where(x >= 0, x, x * SLOPE)


def _psum(accp_ref):
    s = accp_ref[0]
    for w in range(1, NW):
        s = s + accp_ref[w]
    return s


def _tc1_body(degp_ref, x_ref, dinv_ref, u1_ref):
    deg = _psum(degp_ref) + 1.0
    dinv = lax.rsqrt(deg)
    dinv_ref[...] = dinv
    u1_ref[...] = x_ref[...] * dinv


def _tc2_body(accp_ref, dinv_ref, u1_ref, w1_ref, b1_ref, w2_ref, u2_ref):
    dinv = dinv_ref[...]
    agg1 = dinv * _psum(accp_ref) + u1_ref[...] * dinv
    t = jnp.zeros_like(agg1)
    for c in range(CONV):
        h = _lrelu(agg1 * w1_ref[0, c] + b1_ref[c])
        t = t + h * w2_ref[c, 0]
    u2_ref[...] = t * dinv


def _tc3a_body(accp_ref, dinv_ref, u2_ref, b2_ref, v_ref):
    dinv = dinv_ref[...]
    agg2 = dinv * _psum(accp_ref) + u2_ref[...] * dinv + b2_ref[0]
    v_ref[...] = _lrelu(agg2)


def _tc3b_body(v_ref, wf1_ref, bf1_ref, wf2_ref, bf2_ref, out_ref, zacc_ref):
    i = pl.program_id(0)
    part = jnp.dot(v_ref[0], wf1_ref[0], preferred_element_type=jnp.float32)

    @pl.when(i == 0)
    def _():
        zacc_ref[...] = part

    @pl.when(i > 0)
    def _():
        zacc_ref[...] = zacc_ref[...] + part

    @pl.when(i == pl.num_programs(0) - 1)
    def _():
        z = _lrelu(zacc_ref[...] + bf1_ref[...])
        o = jnp.dot(z, wf2_ref[...], preferred_element_type=jnp.float32)
        o = o + bf2_ref[...]
        m = jnp.max(o)
        lse = jnp.log(jnp.sum(jnp.exp(o - m))) + m
        out_ref[...] = o - lse


_NODE = jax.ShapeDtypeStruct((SUB, LANES), jnp.float32)
_VM = pl.BlockSpec(memory_space=pltpu.VMEM)
_SM = pl.BlockSpec(memory_space=pltpu.SMEM)


def _tc1(degp, x2):
    return pl.pallas_call(
        _tc1_body,
        in_specs=[_VM, _VM],
        out_specs=[_VM, _VM],
        out_shape=[_NODE, _NODE],
    )(degp, x2)


def _tc2(accp, dinv2, u12, W1, b1, W2):
    return pl.pallas_call(
        _tc2_body,
        in_specs=[_VM, _VM, _VM, _SM, _SM, _SM],
        out_specs=[_VM],
        out_shape=[_NODE],
    )(accp, dinv2, u12, W1, b1, W2)[0]


def _tc3a(accp, dinv2, u22, b2):
    return pl.pallas_call(
        _tc3a_body,
        in_specs=[_VM, _VM, _VM, _SM],
        out_specs=[_VM],
        out_shape=[_NODE],
    )(accp, dinv2, u22, b2)[0]


def _tc3b(v3, wf1r, bf1, Wf2, bf2):
    return pl.pallas_call(
        _tc3b_body,
        grid=(ROWS,),
        in_specs=[
            pl.BlockSpec((1, 1, COLS), lambda i: (i, 0, 0)),
            pl.BlockSpec((1, COLS, HID), lambda i: (i, 0, 0)),
            pl.BlockSpec((1, HID), lambda i: (0, 0)),
            pl.BlockSpec((HID, OUT), lambda i: (0, 0)),
            pl.BlockSpec((1, OUT), lambda i: (0, 0)),
        ],
        out_specs=pl.BlockSpec((1, OUT), lambda i: (0, 0)),
        out_shape=jax.ShapeDtypeStruct((1, OUT), jnp.float32),
        scratch_shapes=[pltpu.VMEM((1, HID), jnp.float32)],
    )(v3, wf1r, bf1, Wf2, bf2)


def kernel(x, edge_index, W1, b1, W2, b2, Wf1, bf1, Wf2, bf2):
    ei = edge_index.astype(jnp.int32)
    src = ei[0]
    dst = ei[1]
    x2 = x.reshape(SUB, LANES)

    degp = _deg_kernel(dst).reshape(NW, SUB, LANES)
    dinv2, u12 = _tc1(degp, x2)
    acc1p = _edge_kernel(src, dst, u12.reshape(N)).reshape(NW, SUB, LANES)
    u22 = _tc2(acc1p, dinv2, u12, W1, b1, W2)
    acc2p = _edge_kernel(src, dst, u22.reshape(N)).reshape(NW, SUB, LANES)
    v2 = _tc3a(acc2p, dinv2, u22, b2)
    out = _tc3b(v2.reshape(ROWS, 1, COLS), Wf1.reshape(ROWS, COLS, HID),
                bf1.reshape(1, HID), Wf2, bf2.reshape(1, OUT))
    return out.reshape(OUT)


# speedup vs baseline: 172.3388x; 1.4926x over previous
import functools

import jax
import jax.numpy as jnp
from jax import lax
from jax.experimental import pallas as pl
from jax.experimental.pallas import tpu as pltpu
from jax.experimental.pallas import tpu_sc as plsc

N = 50000
E = 1600000
CONV = 32
HID = 512
OUT = 256
SLOPE = 0.01

NC = 2
NS = 16
NW = NC * NS
PER_TILE = E // NW
CHUNK = 2000
NCHUNKS = PER_TILE // CHUNK

ROWS = 125
COLS = 400

_mesh = plsc.VectorSubcoreMesh(core_axis_name="c", subcore_axis_name="s")
_sc_params = pltpu.CompilerParams(needs_layout_passes=False)


def _zero_vmem(acc_v):
    zero16 = jnp.zeros((16,), jnp.float32)

    @plsc.parallel_loop(0, N // 16, unroll=5)
    def _(i):
        acc_v[pl.ds(pl.multiple_of(i * 16, 16), 16)] = zero16


def _edge_slice(hbm, base, k):
    return hbm.at[pl.ds(base + k * CHUNK, CHUNK)]


def _deg_body(dst_hbm, out_hbm, acc_v, bufA, bufB, semA, semB):
    wid = lax.axis_index("s") * NC + lax.axis_index("c")
    _zero_vmem(acc_v)
    base = wid * PER_TILE
    ones16 = jnp.ones((16,), jnp.float32)

    def process(buf):
        @plsc.parallel_loop(0, CHUNK // 16, unroll=5)
        def _(j):
            d = buf[pl.ds(pl.multiple_of(j * 16, 16), 16)]
            plsc.addupdate_scatter(acc_v, [d], ones16)

    pltpu.async_copy(_edge_slice(dst_hbm, base, 0), bufA, semA)

    def pair(p, carry):
        k0 = 2 * p
        pltpu.async_copy(_edge_slice(dst_hbm, base, k0 + 1), bufB, semB)
        pltpu.make_async_copy(_edge_slice(dst_hbm, base, k0), bufA, semA).wait()
        process(bufA)
        pltpu.async_copy(_edge_slice(dst_hbm, base, k0 + 2), bufA, semA)
        pltpu.make_async_copy(_edge_slice(dst_hbm, base, k0 + 1), bufB, semB).wait()
        process(bufB)
        return carry

    lax.fori_loop(0, (NCHUNKS - 1) // 2, pair, 0)
    pltpu.make_async_copy(_edge_slice(dst_hbm, base, NCHUNKS - 1), bufA, semA).wait()
    process(bufA)
    pltpu.sync_copy(acc_v, out_hbm.at[wid])


_deg_kernel = functools.partial(
    pl.kernel,
    mesh=_mesh,
    out_type=jax.ShapeDtypeStruct((NW, N), jnp.float32),
    compiler_params=_sc_params,
    scratch_types=[
        pltpu.VMEM((N,), jnp.float32),
        pltpu.VMEM((CHUNK,), jnp.int32),
        pltpu.VMEM((CHUNK,), jnp.int32),
        pltpu.SemaphoreType.DMA,
        pltpu.SemaphoreType.DMA,
    ],
)(_deg_body)


def _edge_body(src_hbm, dst_hbm, u_hbm, out_hbm, u_v, acc_v,
               srcA, dstA, srcB, dstB, semA, semB):
    wid = lax.axis_index("s") * NC + lax.axis_index("c")
    _zero_vmem(acc_v)
    pltpu.sync_copy(u_hbm, u_v)
    base = wid * PER_TILE

    def start(k, sbuf, dbuf, sem):
        pltpu.async_copy(_edge_slice(src_hbm, base, k), sbuf, sem)
        pltpu.async_copy(_edge_slice(dst_hbm, base, k), dbuf, sem)

    def wait(k, sbuf, dbuf, sem):
        pltpu.make_async_copy(_edge_slice(src_hbm, base, k), sbuf, sem).wait()
        pltpu.make_async_copy(_edge_slice(dst_hbm, base, k), dbuf, sem).wait()

    def process(sbuf, dbuf):
        @plsc.parallel_loop(0, CHUNK // 16, unroll=5)
        def _(j):
            sl = pl.ds(pl.multiple_of(j * 16, 16), 16)
            vals = plsc.load_gather(u_v, [sbuf[sl]])
            plsc.addupdate_scatter(acc_v, [dbuf[sl]], vals)

    start(0, srcA, dstA, semA)

    def pair(p, carry):
        k0 = 2 * p
        start(k0 + 1, srcB, dstB, semB)
        wait(k0, srcA, dstA, semA)
        process(srcA, dstA)
        start(k0 + 2, srcA, dstA, semA)
        wait(k0 + 1, srcB, dstB, semB)
        process(srcB, dstB)
        return carry

    lax.fori_loop(0, (NCHUNKS - 1) // 2, pair, 0)
    wait(NCHUNKS - 1, srcA, dstA, semA)
    process(srcA, dstA)
    pltpu.sync_copy(acc_v, out_hbm.at[wid])


_edge_kernel = functools.partial(
    pl.kernel,
    mesh=_mesh,
    out_type=jax.ShapeDtypeStruct((NW, N), jnp.float32),
    compiler_params=_sc_params,
    scratch_types=[
        pltpu.VMEM((N,), jnp.float32),
        pltpu.VMEM((N,), jnp.float32),
        pltpu.VMEM((CHUNK,), jnp.int32),
        pltpu.VMEM((CHUNK,), jnp.int32),
        pltpu.VMEM((CHUNK,), jnp.int32),
        pltpu.VMEM((CHUNK,), jnp.int32),
        pltpu.SemaphoreType.DMA,
        pltpu.SemaphoreType.DMA,
    ],
)(_edge_body)


def _lrelu(x):
    return jnp.where(x >= 0, x, x * SLOPE)


def _psum(accp_ref):
    s = accp_ref[0]
    for w in range(1, NW):
        s = s + accp_ref[w]
    return s


def _tc1_body(degp_ref, x_ref, dinv_ref, u1_ref):
    deg = _psum(degp_ref) + 1.0
    dinv = lax.rsqrt(deg)
    dinv_ref[...] = dinv
    u1_ref[...] = x_ref[...] * dinv


def _tc2_body(accp_ref, dinv_ref, u1_ref, w1_ref, b1_ref, w2_ref, u2_ref):
    dinv = dinv_ref[...]
    agg1 = dinv * _psum(accp_ref) + u1_ref[...] * dinv
    t = jnp.zeros_like(agg1)
    for c in range(CONV):
        h = _lrelu(agg1 * w1_ref[0, c] + b1_ref[c])
        t = t + h * w2_ref[c, 0]
    u2_ref[...] = t * dinv


def _tc3_body(accp_ref, dinv_ref, u2_ref, b2_ref, wf1_ref, bf1_ref,
              wf2_ref, bf2_ref, out_ref, v_ref, zacc_ref):
    i = pl.program_id(0)

    @pl.when(i == 0)
    def _():
        dinv = dinv_ref[...]
        agg2 = dinv * _psum(accp_ref) + u2_ref[...] * dinv + b2_ref[0]
        v_ref[...] = _lrelu(agg2)
        zacc_ref[...] = jnp.zeros_like(zacc_ref)

    v_blk = v_ref[pl.ds(i, 1), :]
    part = jnp.dot(v_blk, wf1_ref[0], preferred_element_type=jnp.float32)
    zacc_ref[...] = zacc_ref[...] + part

    @pl.when(i == pl.num_programs(0) - 1)
    def _():
        z = _lrelu(zacc_ref[...] + bf1_ref[...])
        o = jnp.dot(z, wf2_ref[...], preferred_element_type=jnp.float32)
        o = o + bf2_ref[...]
        m = jnp.max(o)
        lse = jnp.log(jnp.sum(jnp.exp(o - m))) + m
        out_ref[...] = o - lse


_NODE = jax.ShapeDtypeStruct((ROWS, COLS), jnp.float32)
_VM = pl.BlockSpec(memory_space=pltpu.VMEM)
_SM = pl.BlockSpec(memory_space=pltpu.SMEM)


def _tc1(degp, x2):
    return pl.pallas_call(
        _tc1_body,
        in_specs=[_VM, _VM],
        out_specs=[_VM, _VM],
        out_shape=[_NODE, _NODE],
    )(degp, x2)


def _tc2(accp, dinv2, u12, W1, b1, W2):
    return pl.pallas_call(
        _tc2_body,
        in_specs=[_VM, _VM, _VM, _SM, _SM, _SM],
        out_specs=[_VM],
        out_shape=[_NODE],
    )(accp, dinv2, u12, W1, b1, W2)[0]


def _tc3(accp, dinv2, u22, b2, wf1r, bf1, Wf2, bf2):
    return pl.pallas_call(
        _tc3_body,
        grid=(ROWS,),
        in_specs=[
            pl.BlockSpec((NW, ROWS, COLS), lambda i: (0, 0, 0)),
            pl.BlockSpec((ROWS, COLS), lambda i: (0, 0)),
            pl.BlockSpec((ROWS, COLS), lambda i: (0, 0)),
            pl.BlockSpec(memory_space=pltpu.SMEM),
            pl.BlockSpec((1, COLS, HID), lambda i: (i, 0, 0)),
            pl.BlockSpec((1, HID), lambda i: (0, 0)),
            pl.BlockSpec((HID, OUT), lambda i: (0, 0)),
            pl.BlockSpec((1, OUT), lambda i: (0, 0)),
        ],
        out_specs=pl.BlockSpec((1, OUT), lambda i: (0, 0)),
        out_shape=jax.ShapeDtypeStruct((1, OUT), jnp.float32),
        scratch_shapes=[
            pltpu.VMEM((ROWS, COLS), jnp.float32),
            pltpu.VMEM((1, HID), jnp.float32),
        ],
    )(accp, dinv2, u22, b2, wf1r, bf1, Wf2, bf2)


def kernel(x, edge_index, W1, b1, W2, b2, Wf1, bf1, Wf2, bf2):
    ei = edge_index.astype(jnp.int32)
    src = ei[0]
    dst = ei[1]
    x2 = x.reshape(ROWS, COLS)

    degp = _deg_kernel(dst).reshape(NW, ROWS, COLS)
    dinv2, u12 = _tc1(degp, x2)
    acc1p = _edge_kernel(src, dst, u12.reshape(N)).reshape(NW, ROWS, COLS)
    u22 = _tc2(acc1p, dinv2, u12, W1, b1, W2)
    acc2p = _edge_kernel(src, dst, u22.reshape(N)).reshape(NW, ROWS, COLS)
    out = _tc3(acc2p, dinv2, u22, b2, Wf1.reshape(ROWS, COLS, HID),
               bf1.reshape(1, HID), Wf2, bf2.reshape(1, OUT))
    return out.reshape(OUT)


# speedup vs baseline: 208.0515x; 1.2072x over previous
import functools

import jax
import jax.numpy as jnp
from jax import lax
from jax.experimental import pallas as pl
from jax.experimental.pallas import tpu as pltpu
from jax.experimental.pallas import tpu_sc as plsc

N = 50000
E = 1600000
CONV = 32
HID = 512
OUT = 256
SLOPE = 0.01

NC = 2
NS = 16
NW = NC * NS
CHUNK = 2560
NCHUNKS = E // CHUNK
EXTRA_W = NCHUNKS % NW
BASE_CHUNKS = NCHUNKS // NW

NROWS = 128
COLS = 400
N_PAD = NROWS * COLS
ROUNDS = 4
RSIZE = N_PAD // ROUNDS
RSTRIPE = RSIZE // NS

ROWS = 125

_mesh = plsc.VectorSubcoreMesh(core_axis_name="c", subcore_axis_name="s")
_sc_params = pltpu.CompilerParams(needs_layout_passes=False)


def _zero_vmem(acc_v):
    zero16 = jnp.zeros((16,), jnp.float32)

    @plsc.parallel_loop(0, N_PAD // 16, unroll=10)
    def _(i):
        acc_v[pl.ds(pl.multiple_of(i * 16, 16), 16)] = zero16


def _reduce_partials(sid, acc_v, red_v, shared, sem, on_result):
    for h in range(ROUNDS):
        hbase = h * RSIZE
        pltpu.sync_copy(acc_v.at[pl.ds(hbase, RSIZE)],
                        shared.at[pl.ds(pl.multiple_of(sid * RSIZE, 8), RSIZE)])
        plsc.subcore_barrier()
        off = pl.multiple_of(sid * RSTRIPE, 8)
        for r in range(NS):
            pltpu.async_copy(shared.at[pl.ds(r * RSIZE + off, RSTRIPE)],
                             red_v.at[pl.ds(r * RSTRIPE, RSTRIPE)], sem)
        for r in range(NS):
            pltpu.make_async_copy(shared.at[pl.ds(r * RSIZE + off, RSTRIPE)],
                                  red_v.at[pl.ds(r * RSTRIPE, RSTRIPE)], sem).wait()
        plsc.subcore_barrier()

        @plsc.parallel_loop(0, RSTRIPE // 16, unroll=10)
        def _(j):
            sl = pl.ds(pl.multiple_of(j * 16, 16), 16)
            s = red_v[sl]
            for r in range(1, NS):
                s = s + red_v[pl.ds(pl.multiple_of(r * RSTRIPE + j * 16, 16), 16)]
            red_v[sl] = s

        on_result(hbase + off)


def _chunk_loop(ei_hbm, wid, process, srcA, dstA, srcB, dstB, semA, semB,
                with_src):
    my_n = jnp.where(wid < EXTRA_W, BASE_CHUNKS + 1, BASE_CHUNKS)

    def ei_slice(j, row_base):
        off = pl.multiple_of(row_base + (wid + j * NW) * CHUNK, 128)
        return ei_hbm.at[pl.ds(off, CHUNK)]

    def start(j, sbuf, dbuf, sem):
        @pl.when(j < my_n)
        def _():
            if with_src:
                pltpu.async_copy(ei_slice(j, 0), sbuf, sem)
            pltpu.async_copy(ei_slice(j, E), dbuf, sem)

    def wait(j, sbuf, dbuf, sem):
        if with_src:
            pltpu.make_async_copy(ei_slice(j, 0), sbuf, sem).wait()
        pltpu.make_async_copy(ei_slice(j, E), dbuf, sem).wait()

    start(0, srcA, dstA, semA)

    def pair(p, carry):
        j0 = 2 * p
        start(j0 + 1, srcB, dstB, semB)
        wait(j0, srcA, dstA, semA)
        process(srcA, dstA)
        start(j0 + 2, srcA, dstA, semA)
        wait(j0 + 1, srcB, dstB, semB)
        process(srcB, dstB)
        return carry

    lax.fori_loop(0, my_n // 2, pair, 0)

    @pl.when(my_n % 2 == 1)
    def _():
        wait(my_n - 1, srcA, dstA, semA)
        process(srcA, dstA)


def _fast_rsqrt(d):
    i = plsc.bitcast(d, jnp.int32)
    y = plsc.bitcast(jnp.int32(0x5F3759DF) - (i >> 1), jnp.float32)
    for _ in range(3):
        y = y * (1.5 - 0.5 * d * y * y)
    return y


def _deg_body(ei_hbm, x_hbm, dinv_hbm, u1_hbm, acc_v, red_v, xbuf, dbuf,
              bufA, bufB, shared, semA, semB, semR):
    cid = lax.axis_index("c")
    sid = lax.axis_index("s")
    wid = sid * NC + cid
    _zero_vmem(acc_v)
    ones16 = jnp.ones((16,), jnp.float32)

    def process(sbuf, dbuf):
        @plsc.parallel_loop(0, CHUNK // 16, unroll=10)
        def _(j):
            d = dbuf[pl.ds(pl.multiple_of(j * 16, 16), 16)]
            plsc.addupdate_scatter(acc_v, [d], ones16)

    _chunk_loop(ei_hbm, wid, process, bufA, bufA, bufB, bufB, semA, semB,
                with_src=False)

    def on_result(node_off):
        pltpu.sync_copy(x_hbm.at[pl.ds(node_off, RSTRIPE)], xbuf)

        @plsc.parallel_loop(0, RSTRIPE // 16, unroll=10)
        def _(j):
            sl = pl.ds(pl.multiple_of(j * 16, 16), 16)
            dinv = _fast_rsqrt(red_v[sl] + 1.0)
            dbuf[sl] = dinv
            xbuf[sl] = xbuf[sl] * dinv

        @pl.when(cid == 0)
        def _():
            pltpu.sync_copy(dbuf, dinv_hbm.at[pl.ds(node_off, RSTRIPE)])
            pltpu.sync_copy(xbuf, u1_hbm.at[pl.ds(node_off, RSTRIPE)])

    _reduce_partials(sid, acc_v, red_v, shared, semR, on_result)


_deg_kernel = functools.partial(
    pl.kernel,
    mesh=_mesh,
    out_type=[
        jax.ShapeDtypeStruct((N_PAD,), jnp.float32),
        jax.ShapeDtypeStruct((N_PAD,), jnp.float32),
    ],
    compiler_params=_sc_params,
    scratch_types=[
        pltpu.VMEM((N_PAD,), jnp.float32),
        pltpu.VMEM((RSIZE,), jnp.float32),
        pltpu.VMEM((RSTRIPE,), jnp.float32),
        pltpu.VMEM((RSTRIPE,), jnp.float32),
        pltpu.VMEM((CHUNK,), jnp.int32),
        pltpu.VMEM((CHUNK,), jnp.int32),
        pltpu.VMEM_SHARED((NS * RSIZE,), jnp.float32),
        pltpu.SemaphoreType.DMA,
        pltpu.SemaphoreType.DMA,
        pltpu.SemaphoreType.DMA,
    ],
)(_deg_body)


def _edge_body(ei_hbm, u_hbm, out_hbm, u_v, acc_v,
               srcA, dstA, srcB, dstB, shared, semA, semB, semR):
    cid = lax.axis_index("c")
    sid = lax.axis_index("s")
    wid = sid * NC + cid
    _zero_vmem(acc_v)
    pltpu.sync_copy(u_hbm, u_v)

    def process(sbuf, dbuf):
        @plsc.parallel_loop(0, CHUNK // 16, unroll=10)
        def _(j):
            sl = pl.ds(pl.multiple_of(j * 16, 16), 16)
            vals = plsc.load_gather(u_v, [sbuf[sl]])
            plsc.addupdate_scatter(acc_v, [dbuf[sl]], vals)

    _chunk_loop(ei_hbm, wid, process, srcA, dstA, srcB, dstB, semA, semB,
                with_src=True)

    def on_result(node_off):
        off = pl.multiple_of(cid * N_PAD, 8) + node_off
        pltpu.sync_copy(u_v.at[pl.ds(0, RSTRIPE)], out_hbm.at[pl.ds(off, RSTRIPE)])

    _reduce_partials(sid, acc_v, u_v, shared, semR, on_result)


_edge_kernel = functools.partial(
    pl.kernel,
    mesh=_mesh,
    out_type=jax.ShapeDtypeStruct((NC * N_PAD,), jnp.float32),
    compiler_params=_sc_params,
    scratch_types=[
        pltpu.VMEM((N_PAD,), jnp.float32),
        pltpu.VMEM((N_PAD,), jnp.float32),
        pltpu.VMEM((CHUNK,), jnp.int32),
        pltpu.VMEM((CHUNK,), jnp.int32),
        pltpu.VMEM((CHUNK,), jnp.int32),
        pltpu.VMEM((CHUNK,), jnp.int32),
        pltpu.VMEM_SHARED((NS * RSIZE,), jnp.float32),
        pltpu.SemaphoreType.DMA,
        pltpu.SemaphoreType.DMA,
        pltpu.SemaphoreType.DMA,
    ],
)(_edge_body)


def _lrelu(x):
    return jnp.where(x >= 0, x, x * SLOPE)


def _tc2_body(accp_ref, dinv_ref, u1_ref, w1_ref, b1_ref, w2_ref, u2_ref):
    dinv = dinv_ref[...]
    agg1 = dinv * (accp_ref[0] + accp_ref[1]) + u1_ref[...] * dinv
    t = jnp.zeros_like(agg1)
    for c in range(CONV):
        h = _lrelu(agg1 * w1_ref[0, c] + b1_ref[c])
        t = t + h * w2_ref[c, 0]
    u2_ref[...] = t * dinv


def _tc3_body(accp_ref, dinv_ref, u2_ref, b2_ref, wf1_ref, bf1_ref,
              wf2_ref, bf2_ref, out_ref, v_ref, zacc_ref):
    i = pl.program_id(0)

    @pl.when(i == 0)
    def _():
        dinv = dinv_ref[...]
        agg2 = dinv * (accp_ref[0] + accp_ref[1]) + u2_ref[...] * dinv + b2_ref[0]
        v_ref[...] = _lrelu(agg2)
        zacc_ref[...] = jnp.zeros_like(zacc_ref)

    v_blk = v_ref[pl.ds(i, 1), :]
    part = jnp.dot(v_blk, wf1_ref[0], preferred_element_type=jnp.float32)
    zacc_ref[...] = zacc_ref[...] + part

    @pl.when(i == pl.num_programs(0) - 1)
    def _():
        z = _lrelu(zacc_ref[...] + bf1_ref[...])
        o = jnp.dot(z, wf2_ref[...], preferred_element_type=jnp.float32)
        o = o + bf2_ref[...]
        m = jnp.max(o)
        lse = jnp.log(jnp.sum(jnp.exp(o - m))) + m
        out_ref[...] = o - lse


_VM = pl.BlockSpec(memory_space=pltpu.VMEM)
_SM = pl.BlockSpec(memory_space=pltpu.SMEM)


def _tc2(accp, dinv2, u12, W1, b1, W2):
    return pl.pallas_call(
        _tc2_body,
        in_specs=[_VM, _VM, _VM, _SM, _SM, _SM],
        out_specs=[_VM],
        out_shape=[jax.ShapeDtypeStruct((NROWS, COLS), jnp.float32)],
    )(accp, dinv2, u12, W1, b1, W2)[0]


def _tc3(accp, dinv2, u22, b2, wf1r, bf1, Wf2, bf2):
    return pl.pallas_call(
        _tc3_body,
        grid=(ROWS,),
        in_specs=[
            pl.BlockSpec((NC, NROWS, COLS), lambda i: (0, 0, 0)),
            pl.BlockSpec((NROWS, COLS), lambda i: (0, 0)),
            pl.BlockSpec((NROWS, COLS), lambda i: (0, 0)),
            pl.BlockSpec(memory_space=pltpu.SMEM),
            pl.BlockSpec((1, COLS, HID), lambda i: (i, 0, 0)),
            pl.BlockSpec((1, HID), lambda i: (0, 0)),
            pl.BlockSpec((HID, OUT), lambda i: (0, 0)),
            pl.BlockSpec((1, OUT), lambda i: (0, 0)),
        ],
        out_specs=pl.BlockSpec((1, OUT), lambda i: (0, 0)),
        out_shape=jax.ShapeDtypeStruct((1, OUT), jnp.float32),
        scratch_shapes=[
            pltpu.VMEM((NROWS, COLS), jnp.float32),
            pltpu.VMEM((1, HID), jnp.float32),
        ],
    )(accp, dinv2, u22, b2, wf1r, bf1, Wf2, bf2)


def kernel(x, edge_index, W1, b1, W2, b2, Wf1, bf1, Wf2, bf2):
    ei = edge_index.astype(jnp.int32).reshape(2 * E)
    xflat = jnp.pad(x.reshape(N), (0, N_PAD - N))

    dinv, u1 = _deg_kernel(ei, xflat)
    acc1p = _edge_kernel(ei, u1)
    u22 = _tc2(acc1p.reshape(NC, NROWS, COLS), dinv.reshape(NROWS, COLS),
               u1.reshape(NROWS, COLS), W1, b1, W2)
    acc2p = _edge_kernel(ei, u22.reshape(N_PAD))
    out = _tc3(acc2p.reshape(NC, NROWS, COLS), dinv.reshape(NROWS, COLS),
               u22, b2, Wf1.reshape(ROWS, COLS, HID),
               bf1.reshape(1, HID), Wf2, bf2.reshape(1, OUT))
    return out.reshape(OUT)
